# baseline (speedup 1.0000x reference)
import jax
import jax.numpy as jnp
from jax.experimental import pallas as pl
from jax.experimental.pallas import tpu as pltpu

_TRIM = 32
_N = 256
_EPS = 1e-12
_W = 128
_NV = _N // 8


_PAIR = 2


def _chunk(x_ref, o_ref, c):
    bases = [c * (_PAIR * _W) + q * _W for q in range(_PAIR)]
    v = [x_ref[i * 8:(i + 1) * 8, pl.ds(b, _W)]
         for i in range(_NV) for b in bases]

    s = jax.lax.broadcasted_iota(jnp.int32, (8, 1), 0)
    upper_t = {1: (s & 1) != 0, 2: (s & 2) != 0, 4: (s & 4) != 0}

    def cross(jv, k):
        kb = k.bit_length() - 1
        for a in range(_NV):
            if a & jv:
                continue
            for q in range(_PAIR):
                ia = a * _PAIR + q
                ib = (a ^ jv) * _PAIR + q
                mn = jnp.minimum(v[ia], v[ib])
                mx = jnp.maximum(v[ia], v[ib])
                if kb <= 4 and (a & k) != 0:
                    v[ia], v[ib] = mx, mn
                else:
                    v[ia], v[ib] = mn, mx

    def sublane(t):
        up = upper_t[t]
        for a in range(_NV * _PAIR):
            y = v[a]
            if t == 4:
                partner = pltpu.roll(y, 4, axis=0)
            else:
                partner = jnp.where(up,
                                    pltpu.roll(y, t, axis=0),
                                    pltpu.roll(y, 8 - t, axis=0))
            v[a] = jnp.where(up, jnp.maximum(y, partner),
                             jnp.minimum(y, partner))

    def flip(mask):
        sgn = jnp.where(mask, -1.0, 1.0).astype(jnp.float32)
        for a in range(_NV * _PAIR):
            v[a] = v[a] * sgn

    strides = {5: 4, 6: 1, 7: 2}
    f4 = (s & 4) != 0
    f1 = (s & 1) != 0
    f2 = (s & 2) != 0
    flips = {32: f4,
             64: f4 != f1,
             128: f1 != f2,
             256: f2}

    k = 2
    while k <= _N:
        if k in flips:
            flip(flips[k])
        j = k // 2
        while j >= 1:
            b = j.bit_length() - 1
            if b <= 4:
                if k < _N:
                    cross(1 << b, k)
            else:
                sublane(strides[b])
            j //= 2
        k *= 2

    def tree_sum(lst):
        while len(lst) > 1:
            lst = [lst[i] + lst[i + 1] for i in range(0, len(lst) - 1, 2)] + (
                [lst[-1]] if len(lst) % 2 else [])
        return lst[0]

    keep = (s >= 1) & (s <= 6)
    for q in range(_PAIR):
        tot = tree_sum([v[i * _PAIR + q] for i in range(_NV)])
        tsum = jnp.sum(jnp.where(keep, tot, 0.0), axis=0, keepdims=True)
        u = tsum / (_N - 2 * _TRIM)
        o_ref[0:1, pl.ds(bases[q], _W)] = jnp.log(jnp.maximum(u, _EPS))


_CHUNKS = 16
_WB = _W * _CHUNKS


def _trim_body(x_ref, o_ref):
    def body(c, carry):
        _chunk(x_ref, o_ref, c)
        return carry

    jax.lax.fori_loop(0, _CHUNKS // _PAIR, body, 0)


@jax.jit
def kernel(x):
    n, d = x.shape
    grid = pl.cdiv(d, _WB)
    out = pl.pallas_call(
        _trim_body,
        grid=(grid,),
        in_specs=[pl.BlockSpec((n, _WB), lambda i: (0, i))],
        out_specs=pl.BlockSpec((1, _WB), lambda i: (0, i)),
        out_shape=jax.ShapeDtypeStruct((1, d), x.dtype),
        compiler_params=pltpu.CompilerParams(
            dimension_semantics=("parallel",),
        ),
    )(x)
    return out.reshape(d)


# speedup vs baseline: 1.0630x; 1.0630x over previous
import jax
import jax.numpy as jnp
from jax.experimental import pallas as pl
from jax.experimental.pallas import tpu as pltpu

_TRIM = 32
_N = 256
_EPS = 1e-12
_W = 128
_NV = _N // 8


_PAIR = 1


def _chunk(x_ref, o_ref, c):
    bases = [c * (_PAIR * _W) + q * _W for q in range(_PAIR)]
    v = [x_ref[i * 8:(i + 1) * 8, pl.ds(b, _W)]
         for i in range(_NV) for b in bases]

    s = jax.lax.broadcasted_iota(jnp.int32, (8, 1), 0)
    upper_t = {1: (s & 1) != 0, 2: (s & 2) != 0, 4: (s & 4) != 0}

    def cross(jv, k):
        kb = k.bit_length() - 1
        for a in range(_NV):
            if a & jv:
                continue
            for q in range(_PAIR):
                ia = a * _PAIR + q
                ib = (a ^ jv) * _PAIR + q
                mn = jnp.minimum(v[ia], v[ib])
                mx = jnp.maximum(v[ia], v[ib])
                if kb <= 4 and (a & k) != 0:
                    v[ia], v[ib] = mx, mn
                else:
                    v[ia], v[ib] = mn, mx

    def sublane(t):
        up = upper_t[t]
        for a in range(_NV * _PAIR):
            y = v[a]
            if t == 4:
                partner = pltpu.roll(y, 4, axis=0)
            else:
                partner = jnp.where(up,
                                    pltpu.roll(y, t, axis=0),
                                    pltpu.roll(y, 8 - t, axis=0))
            v[a] = jnp.where(up, jnp.maximum(y, partner),
                             jnp.minimum(y, partner))

    def flip(mask):
        sgn = jnp.where(mask, -1.0, 1.0).astype(jnp.float32)
        for a in range(_NV * _PAIR):
            v[a] = v[a] * sgn

    strides = {5: 4, 6: 1, 7: 2}
    f4 = (s & 4) != 0
    f1 = (s & 1) != 0
    f2 = (s & 2) != 0
    flips = {32: f4,
             64: f4 != f1,
             128: f1 != f2,
             256: f2}

    k = 2
    while k <= _N:
        if k in flips:
            flip(flips[k])
        j = k // 2
        while j >= 1:
            b = j.bit_length() - 1
            if b <= 4:
                if k < _N:
                    cross(1 << b, k)
            else:
                sublane(strides[b])
            j //= 2
        k *= 2

    def tree_sum(lst):
        while len(lst) > 1:
            lst = [lst[i] + lst[i + 1] for i in range(0, len(lst) - 1, 2)] + (
                [lst[-1]] if len(lst) % 2 else [])
        return lst[0]

    keep = (s >= 1) & (s <= 6)
    for q in range(_PAIR):
        tot = tree_sum([v[i * _PAIR + q] for i in range(_NV)])
        tsum = jnp.sum(jnp.where(keep, tot, 0.0), axis=0, keepdims=True)
        u = tsum / (_N - 2 * _TRIM)
        o_ref[0:1, pl.ds(bases[q], _W)] = jnp.log(jnp.maximum(u, _EPS))


_CHUNKS = 4
_WB = _W * _CHUNKS


def _trim_body(x_ref, o_ref):
    for c in range(_CHUNKS // _PAIR):
        _chunk(x_ref, o_ref, c)


@jax.jit
def kernel(x):
    n, d = x.shape
    grid = pl.cdiv(d, _WB)
    out = pl.pallas_call(
        _trim_body,
        grid=(grid,),
        in_specs=[pl.BlockSpec((n, _WB), lambda i: (0, i))],
        out_specs=pl.BlockSpec((1, _WB), lambda i: (0, i)),
        out_shape=jax.ShapeDtypeStruct((1, d), x.dtype),
        compiler_params=pltpu.CompilerParams(
            dimension_semantics=("parallel",),
        ),
    )(x)
    return out.reshape(d)


# speedup vs baseline: 1.1324x; 1.0653x over previous
import jax
import jax.numpy as jnp
from jax.experimental import pallas as pl
from jax.experimental.pallas import tpu as pltpu

_TRIM = 32
_N = 256
_EPS = 1e-12
_W = 128
_NV = _N // 8


_PAIR = 1


def _chunk(x_ref, o_ref, c):
    bases = [c * (_PAIR * _W) + q * _W for q in range(_PAIR)]
    v = [x_ref[i * 8:(i + 1) * 8, pl.ds(b, _W)]
         for i in range(_NV) for b in bases]

    s = jax.lax.broadcasted_iota(jnp.int32, (8, 1), 0)
    upper_t = {1: (s & 1) != 0, 2: (s & 2) != 0, 4: (s & 4) != 0}

    def cross(jv, k):
        kb = k.bit_length() - 1
        for a in range(_NV):
            if a & jv:
                continue
            for q in range(_PAIR):
                ia = a * _PAIR + q
                ib = (a ^ jv) * _PAIR + q
                mn = jnp.minimum(v[ia], v[ib])
                mx = jnp.maximum(v[ia], v[ib])
                if kb <= 4 and (a & k) != 0:
                    v[ia], v[ib] = mx, mn
                else:
                    v[ia], v[ib] = mn, mx

    def sublane(t):
        up = upper_t[t]
        for a in range(_NV * _PAIR):
            y = v[a]
            if t == 4:
                partner = pltpu.roll(y, 4, axis=0)
                v[a] = jnp.where(up, jnp.maximum(y, partner),
                                 jnp.minimum(y, partner))
            else:
                v[a] = jnp.where(up,
                                 jnp.maximum(y, pltpu.roll(y, t, axis=0)),
                                 jnp.minimum(y, pltpu.roll(y, 8 - t, axis=0)))

    def flip(mask):
        sgn = jnp.where(mask, -1.0, 1.0).astype(jnp.float32)
        for a in range(_NV * _PAIR):
            v[a] = v[a] * sgn

    strides = {5: 4, 6: 1, 7: 2}
    f4 = (s & 4) != 0
    f1 = (s & 1) != 0
    f2 = (s & 2) != 0
    flips = {32: f4,
             64: f4 != f1,
             128: f1 != f2,
             256: f2}

    k = 2
    while k <= _N:
        if k in flips:
            flip(flips[k])
        j = k // 2
        while j >= 1:
            b = j.bit_length() - 1
            if b <= 4:
                if k < _N:
                    cross(1 << b, k)
            else:
                sublane(strides[b])
            j //= 2
        k *= 2

    def tree_sum(lst):
        while len(lst) > 1:
            lst = [lst[i] + lst[i + 1] for i in range(0, len(lst) - 1, 2)] + (
                [lst[-1]] if len(lst) % 2 else [])
        return lst[0]

    keep = (s >= 1) & (s <= 6)
    for q in range(_PAIR):
        tot = tree_sum([v[i * _PAIR + q] for i in range(_NV)])
        tsum = jnp.sum(jnp.where(keep, tot, 0.0), axis=0, keepdims=True)
        u = tsum / (_N - 2 * _TRIM)
        o_ref[0:1, pl.ds(bases[q], _W)] = jnp.log(jnp.maximum(u, _EPS))


_CHUNKS = 8
_WB = _W * _CHUNKS


def _trim_body(x_ref, o_ref):
    for c in range(_CHUNKS // _PAIR):
        _chunk(x_ref, o_ref, c)


@jax.jit
def kernel(x):
    n, d = x.shape
    grid = pl.cdiv(d, _WB)
    out = pl.pallas_call(
        _trim_body,
        grid=(grid,),
        in_specs=[pl.BlockSpec((n, _WB), lambda i: (0, i))],
        out_specs=pl.BlockSpec((1, _WB), lambda i: (0, i)),
        out_shape=jax.ShapeDtypeStruct((1, d), x.dtype),
        compiler_params=pltpu.CompilerParams(
            dimension_semantics=("parallel",),
        ),
    )(x)
    return out.reshape(d)
